# baseline (speedup 1.0000x reference)
import functools

import jax
import jax.numpy as jnp
from jax import lax
from jax.experimental import pallas as pl
from jax.experimental.pallas import tpu as pltpu
from jax.experimental.pallas import tpu_sc as plsc

DEPTH = 64
NUM_TOK = 8192
NUM_WORKERS = 32
TOK_PER_W = NUM_TOK // NUM_WORKERS
SEG_PER_ROW = 2048 // TOK_PER_W
CHUNK = 128
NCHUNK = TOK_PER_W // CHUNK
LANES = 16


def _emb_body(idx_hbm, tok_hbm, pos_hbm, out_hbm, idx_v, idx2_v, buf_v,
              pos_v, out_v, gsem, psem):
    wid = lax.axis_index("s") * 2 + lax.axis_index("c")
    tok_base = wid * TOK_PER_W
    pos_base = (wid % SEG_PER_ROW) * TOK_PER_W

    pltpu.sync_copy(idx_hbm.at[pl.ds(tok_base, TOK_PER_W)], idx_v)

    pcopy = pltpu.async_copy(pos_hbm.at[pl.ds(pos_base, TOK_PER_W)], pos_v,
                             psem)

    for c in range(NCHUNK):
        for j in range(CHUNK // LANES):
            sl = pl.ds(j * LANES, LANES)
            src = pl.ds(c * CHUNK + j * LANES, LANES)
            idx2_v[c, sl] = jax.lax.shift_right_logical(idx_v[src], 1)

    gcopies = [
        pltpu.async_copy(tok_hbm.at[idx2_v.at[c]],
                         buf_v.at[pl.ds(c * CHUNK, CHUNK)], gsem)
        for c in range(NCHUNK)
    ]
    for cp in gcopies:
        cp.wait()
    pcopy.wait()

    def sel_add(i, carry):
        tok = plsc.load_gather(idx_v, [jnp.full((LANES,), i, jnp.int32)])
        odd = (tok & 1) == 1
        for k in range(DEPTH // LANES):
            sl = pl.ds(k * LANES, LANES)
            lo = buf_v[i, sl]
            hi = buf_v[i, pl.ds(DEPTH + k * LANES, LANES)]
            out_v[i, sl] = jnp.where(odd, hi, lo) + pos_v[i, sl]
        return carry

    lax.fori_loop(0, TOK_PER_W, sel_add, 0)

    pltpu.sync_copy(out_v, out_hbm.at[pl.ds(tok_base, TOK_PER_W)])


_emb_call = functools.partial(
    pl.kernel,
    mesh=plsc.VectorSubcoreMesh(core_axis_name="c", subcore_axis_name="s"),
    out_type=jax.ShapeDtypeStruct((NUM_TOK, DEPTH), jnp.float32),
    scratch_types=[
        pltpu.VMEM((TOK_PER_W,), jnp.int32),
        pltpu.VMEM((NCHUNK, CHUNK), jnp.int32),
        pltpu.VMEM((TOK_PER_W, 2 * DEPTH), jnp.float32),
        pltpu.VMEM((TOK_PER_W, DEPTH), jnp.float32),
        pltpu.VMEM((TOK_PER_W, DEPTH), jnp.float32),
        pltpu.SemaphoreType.DMA,
        pltpu.SemaphoreType.DMA,
    ],
    compiler_params=pltpu.CompilerParams(needs_layout_passes=False),
)(_emb_body)


def kernel(x, tok_table, pos_table):
    b, s = x.shape
    xf = x.reshape(NUM_TOK).astype(jnp.int32)
    tok2 = tok_table.reshape(tok_table.shape[0] // 2, 2 * DEPTH)
    out = _emb_call(xf, tok2, pos_table)
    return out.reshape(b, s, DEPTH)


# speedup vs baseline: 1.3332x; 1.3332x over previous
import functools

import jax
import jax.numpy as jnp
from jax import lax
from jax.experimental import pallas as pl
from jax.experimental.pallas import tpu as pltpu
from jax.experimental.pallas import tpu_sc as plsc

DEPTH = 64
NUM_TOK = 8192
NUM_WORKERS = 32
TOK_PER_W = NUM_TOK // NUM_WORKERS
SEG_PER_ROW = 2048 // TOK_PER_W
LANES = 16
NGROUP = TOK_PER_W // LANES


def _emb_body(idx_hbm, tok_hbm, pos_hbm, out_hbm, idx_v, buf_v, pos_v,
              gsem, psem):
    wid = lax.axis_index("s") * 2 + lax.axis_index("c")
    tok_base = wid * TOK_PER_W
    pos_base = (wid % SEG_PER_ROW) * TOK_PER_W

    pltpu.sync_copy(idx_hbm.at[pl.ds(tok_base, TOK_PER_W)], idx_v)

    pcopy = pltpu.async_copy(pos_hbm.at[pl.ds(pos_base, TOK_PER_W)], pos_v,
                             psem)

    pending = []
    copies = []
    for g in range(NGROUP):
        toks = idx_v[pl.ds(g * LANES, LANES)]
        for j in range(LANES):
            row = toks[j]
            i = g * LANES + j
            copies.append(
                pltpu.async_copy(tok_hbm.at[pl.ds(row, 1)],
                                 buf_v.at[pl.ds(i, 1)], gsem))
        if pending:
            for cp in pending:
                cp.wait()
        pending = copies
        copies = []
    for cp in pending:
        cp.wait()
    pcopy.wait()

    def add_row(i, carry):
        for k in range(DEPTH // LANES):
            sl = pl.ds(k * LANES, LANES)
            buf_v[i, sl] = buf_v[i, sl] + pos_v[i, sl]
        return carry

    lax.fori_loop(0, TOK_PER_W, add_row, 0)

    pltpu.sync_copy(buf_v, out_hbm.at[pl.ds(tok_base, TOK_PER_W)])


_emb_call = functools.partial(
    pl.kernel,
    mesh=plsc.VectorSubcoreMesh(core_axis_name="c", subcore_axis_name="s"),
    out_type=jax.ShapeDtypeStruct((NUM_TOK, DEPTH), jnp.float32),
    scratch_types=[
        pltpu.VMEM((TOK_PER_W,), jnp.int32),
        pltpu.VMEM((TOK_PER_W, DEPTH), jnp.float32),
        pltpu.VMEM((TOK_PER_W, DEPTH), jnp.float32),
        pltpu.SemaphoreType.DMA,
        pltpu.SemaphoreType.DMA,
    ],
    compiler_params=pltpu.CompilerParams(needs_layout_passes=False),
)(_emb_body)


def kernel(x, tok_table, pos_table):
    b, s = x.shape
    xf = x.reshape(NUM_TOK).astype(jnp.int32)
    out = _emb_call(xf, tok_table, pos_table)
    return out.reshape(b, s, DEPTH)


# speedup vs baseline: 2.3111x; 1.7334x over previous
import functools

import jax
import jax.numpy as jnp
from jax import lax
from jax.experimental import pallas as pl
from jax.experimental.pallas import tpu as pltpu
from jax.experimental.pallas import tpu_sc as plsc

VOCAB = 100000
DEPTH = 64
BATCH = 4
SEQ = 2048
NUM_TOK = BATCH * SEQ
LANES = 16
D_PER_W = DEPTH // 32


def _emb_body(idx_hbm, tok_hbm, pos_hbm, out_hbm, idx_v, row_v, pos_v, ob_v,
              rsem, psem):
    wid = lax.axis_index("s") * 2 + lax.axis_index("c")
    d0 = wid * D_PER_W

    rcopy = pltpu.async_copy(tok_hbm.at[d0], row_v, rsem)
    pltpu.sync_copy(idx_hbm, idx_v)
    pcopy = pltpu.async_copy(pos_hbm.at[pl.ds(d0, D_PER_W)], pos_v, psem)
    pcopy.wait()

    for t in range(D_PER_W):
        rcopy.wait()

        def chunk(c, carry):
            sl = pl.ds(c * LANES, LANES)
            toks = idx_v[sl]
            vals = plsc.load_gather(row_v, [toks])
            s16 = lax.rem(c, SEQ // LANES) * LANES
            ob_v[sl] = vals + pos_v[t, pl.ds(s16, LANES)]
            return carry

        lax.fori_loop(0, NUM_TOK // LANES, chunk, 0)

        if t + 1 < D_PER_W:
            rcopy = pltpu.async_copy(tok_hbm.at[d0 + t + 1], row_v, rsem)

        for b in range(BATCH):
            pltpu.sync_copy(ob_v.at[pl.ds(b * SEQ, SEQ)],
                            out_hbm.at[b * DEPTH + d0 + t])


_emb_call = functools.partial(
    pl.kernel,
    mesh=plsc.VectorSubcoreMesh(core_axis_name="c", subcore_axis_name="s"),
    out_type=jax.ShapeDtypeStruct((BATCH * DEPTH, SEQ), jnp.float32),
    scratch_types=[
        pltpu.VMEM((NUM_TOK,), jnp.int32),
        pltpu.VMEM((VOCAB,), jnp.float32),
        pltpu.VMEM((D_PER_W, SEQ), jnp.float32),
        pltpu.VMEM((NUM_TOK,), jnp.float32),
        pltpu.SemaphoreType.DMA,
        pltpu.SemaphoreType.DMA,
    ],
    compiler_params=pltpu.CompilerParams(needs_layout_passes=False),
)(_emb_body)


def kernel(x, tok_table, pos_table):
    b, s = x.shape
    xf = x.reshape(NUM_TOK).astype(jnp.int32)
    out = _emb_call(xf, tok_table.T, pos_table.T)
    return out.reshape(b, DEPTH, s).transpose(0, 2, 1)


# speedup vs baseline: 3.1138x; 1.3473x over previous
import functools

import jax
import jax.numpy as jnp
from jax import lax
from jax.experimental import pallas as pl
from jax.experimental.pallas import tpu as pltpu
from jax.experimental.pallas import tpu_sc as plsc

VOCAB = 100000
DEPTH = 64
BATCH = 4
SEQ = 2048
NUM_TOK = BATCH * SEQ
LANES = 16
D_PER_W = DEPTH // 32


def _emb_body(idx_hbm, tok_hbm, pos_hbm, out_hbm, idx_v, row_v, pos_v, ob_v,
              rsem, psem, osem):
    wid = lax.axis_index("s") * 2 + lax.axis_index("c")
    d0 = wid * D_PER_W

    rcopy = pltpu.async_copy(tok_hbm.at[d0], row_v, rsem)
    pltpu.sync_copy(idx_hbm, idx_v)
    pcopy = pltpu.async_copy(pos_hbm.at[pl.ds(d0, D_PER_W)], pos_v, psem)
    pcopy.wait()

    ocopies = []
    for t in range(D_PER_W):
        rcopy.wait()
        for cp in ocopies:
            cp.wait()
        ocopies = []

        for b in range(BATCH):

            @functools.partial(
                plsc.parallel_loop, 0, SEQ // LANES, unroll=4)
            def chunk(c, _b=b):
                sl = pl.ds(_b * SEQ + c * LANES, LANES)
                toks = idx_v[sl]
                vals = plsc.load_gather(row_v, [toks])
                ob_v[sl] = vals + pos_v[t, pl.ds(c * LANES, LANES)]

        if t + 1 < D_PER_W:
            rcopy = pltpu.async_copy(tok_hbm.at[d0 + t + 1], row_v, rsem)

        ocopies = [
            pltpu.async_copy(ob_v.at[pl.ds(b * SEQ, SEQ)],
                             out_hbm.at[b * DEPTH + d0 + t], osem)
            for b in range(BATCH)
        ]
    for cp in ocopies:
        cp.wait()


_emb_call = functools.partial(
    pl.kernel,
    mesh=plsc.VectorSubcoreMesh(core_axis_name="c", subcore_axis_name="s"),
    out_type=jax.ShapeDtypeStruct((BATCH * DEPTH, SEQ), jnp.float32),
    scratch_types=[
        pltpu.VMEM((NUM_TOK,), jnp.int32),
        pltpu.VMEM((VOCAB,), jnp.float32),
        pltpu.VMEM((D_PER_W, SEQ), jnp.float32),
        pltpu.VMEM((NUM_TOK,), jnp.float32),
        pltpu.SemaphoreType.DMA,
        pltpu.SemaphoreType.DMA,
        pltpu.SemaphoreType.DMA,
    ],
    compiler_params=pltpu.CompilerParams(needs_layout_passes=False),
)(_emb_body)


def kernel(x, tok_table, pos_table):
    b, s = x.shape
    xf = x.reshape(NUM_TOK).astype(jnp.int32)
    out = _emb_call(xf, tok_table.T, pos_table.T)
    return out.reshape(b, DEPTH, s).transpose(0, 2, 1)
